# Initial kernel scaffold; baseline (speedup 1.0000x reference)
import jax
import jax.numpy as jnp
from jax.experimental import pallas as pl


def kernel(x, table, freq_scale):
    raise NotImplementedError("write your pallas kernel here")


# speedup vs baseline: 1.0974x; 1.0974x over previous
import functools

import jax
import jax.numpy as jnp
from jax import lax
from jax.experimental import pallas as pl
from jax.experimental.pallas import tpu as pltpu
from jax.experimental.pallas import tpu_sc as plsc

_INFO = plsc.get_sparse_core_info()
_NC = _INFO.num_cores
_NS = _INFO.num_subcores
_NW = _NC * _NS

_CHUNK = 1024
_SUB = 128


def _msqrt_tc(freq_scale):
    n = freq_scale.shape[0]
    s2 = freq_scale.reshape(1000, n // 1000)

    def body(s_ref, o_ref):
        o_ref[...] = jnp.sqrt(jnp.clip(s_ref[...], 0.1, 10.0))

    out = pl.pallas_call(
        body,
        out_shape=jax.ShapeDtypeStruct(s2.shape, jnp.float32),
    )(s2)
    return out.reshape(n)


def _sc_gather_scale(idx_flat, table, msqrt, n_rows, dim):
    per_w = n_rows // _NW
    n_chunks = per_w // _CHUNK
    n_sub = _CHUNK // _SUB
    mesh = plsc.VectorSubcoreMesh(core_axis_name="c", subcore_axis_name="s")

    @functools.partial(
        pl.kernel,
        mesh=mesh,
        out_type=jax.ShapeDtypeStruct((n_rows, dim), jnp.float32),
        scratch_types=[
            pltpu.VMEM((_CHUNK,), jnp.int32),
            pltpu.VMEM((_CHUNK, dim), jnp.float32),
            pltpu.VMEM((_CHUNK,), jnp.float32),
            pltpu.SemaphoreType.DMA,
            pltpu.SemaphoreType.DMA,
        ],
        compiler_params=pltpu.CompilerParams(use_tc_tiling_on_sc=False),
    )
    def k(idx_hbm, table_hbm, m_hbm, out_hbm, idx_v, rows_v, m_v, sem_r, sem_m):
        wid = lax.axis_index("s") * _NC + lax.axis_index("c")
        base = wid * per_w

        def chunk_body(g, carry):
            off = base + g * _CHUNK
            pltpu.sync_copy(idx_hbm.at[pl.ds(off, _CHUNK)], idx_v)
            for u in range(n_sub):
                sl = pl.ds(u * _SUB, _SUB)
                pltpu.async_copy(
                    table_hbm.at[idx_v.at[sl]], rows_v.at[sl], sem_r)
                pltpu.async_copy(m_hbm.at[idx_v.at[sl]], m_v.at[sl], sem_m)
            for u in range(n_sub):
                sl = pl.ds(u * _SUB, _SUB)
                pltpu.make_async_copy(
                    table_hbm.at[idx_v.at[sl]], rows_v.at[sl], sem_r).wait()
                pltpu.make_async_copy(
                    m_hbm.at[idx_v.at[sl]], m_v.at[sl], sem_m).wait()

            def row_body(r, c):
                mvec = m_v[pl.ds(r * 16, 16)]
                for j in range(16):
                    i = r * 16 + j
                    s = mvec[j]
                    rows_v[i, pl.ds(0, 16)] = rows_v[i, pl.ds(0, 16)] * s
                    rows_v[i, pl.ds(16, 16)] = rows_v[i, pl.ds(16, 16)] * s
                return c

            lax.fori_loop(0, _CHUNK // 16, row_body, 0)
            pltpu.sync_copy(rows_v, out_hbm.at[pl.ds(off, _CHUNK)])
            return carry

        lax.fori_loop(0, n_chunks, chunk_body, 0)

    return k(idx_flat, table, msqrt)


def kernel(x, table, freq_scale):
    b, h = x.shape
    dim = table.shape[1]
    n_rows = b * h
    msqrt = _msqrt_tc(freq_scale)
    out = _sc_gather_scale(x.reshape(n_rows), table, msqrt, n_rows, dim)
    return out.reshape(b, h, dim)


# speedup vs baseline: 2.0817x; 1.8970x over previous
import functools

import jax
import jax.numpy as jnp
from jax import lax
from jax.experimental import pallas as pl
from jax.experimental.pallas import tpu as pltpu
from jax.experimental.pallas import tpu_sc as plsc

_INFO = plsc.get_sparse_core_info()
_NC = _INFO.num_cores
_NS = _INFO.num_subcores
_NW = _NC * _NS

_LANE = 128
_NBUF = 4
_NRING = 8


def _scale_table_tc(table_t, freq_scale):
    dim, v = table_t.shape
    r = 4096
    n_steps = pl.cdiv(v, r)
    v_pad = n_steps * r
    fold = 128 // dim

    def body(tab_ref, fs_ref, o_ref):
        s = jnp.sqrt(jnp.clip(fs_ref[...], 0.1, 10.0))
        y = (tab_ref[...] * s[None, :]).T
        y4 = y.reshape(r // fold, fold, dim)
        for k in range(fold):
            o_ref[:, pl.ds(k * dim, dim)] = y4[:, k, :]

    return pl.pallas_call(
        body,
        grid=(n_steps,),
        in_specs=[
            pl.BlockSpec((dim, r), lambda i: (0, i)),
            pl.BlockSpec((r,), lambda i: (i,)),
        ],
        out_specs=pl.BlockSpec((r // fold, fold * dim), lambda i: (i, 0)),
        out_shape=jax.ShapeDtypeStruct((v_pad // fold, fold * dim), jnp.float32),
    )(table_t, freq_scale)


def _repack_idx_tc(x_t):
    h, b = x_t.shape

    def body(x_ref, o_ref):
        o_ref[...] = x_ref[...].reshape(h, b // _LANE, _LANE)

    return pl.pallas_call(
        body,
        out_shape=jax.ShapeDtypeStruct((h, b // _LANE, _LANE), jnp.int32),
    )(x_t)


def _sc_gather(idx_tiles, scaled_lin, hist, batch, dim):
    n_bh = batch // _LANE
    n_tiles = hist * n_bh
    per_w = n_tiles // _NW
    n_groups = per_w // _NBUF
    dim_hi = dim // 8
    mesh = plsc.VectorSubcoreMesh(core_axis_name="c", subcore_axis_name="s")

    scratch = [pltpu.VMEM((per_w, _LANE), jnp.int32)]
    for _ in range(_NRING):
        scratch.append(pltpu.VMEM((_LANE, dim), jnp.float32))
    for _ in range(2):
        scratch.append(
            pltpu.VMEM((dim_hi, _NBUF, 8, _LANE), jnp.float32))
    scratch += [
        pltpu.SemaphoreType.DMA,
        pltpu.SemaphoreType.DMA,
    ]

    @functools.partial(
        pl.kernel,
        mesh=mesh,
        out_type=jax.ShapeDtypeStruct(
            (hist, dim_hi, n_bh, 8, _LANE), jnp.float32),
        scratch_types=scratch,
        compiler_params=pltpu.CompilerParams(
            use_tc_tiling_on_sc=False, needs_layout_passes=False),
    )
    def k(idx_hbm, tab_hbm, out_hbm, idx_all, *refs):
        rows_bufs = refs[:_NRING]
        stages = refs[_NRING:_NRING + 2]
        sem_r, sem_o = refs[_NRING + 2:]
        wid = lax.axis_index("s") * _NC + lax.axis_index("c")
        tile0 = wid * per_w
        lastl = per_w - 1

        pltpu.sync_copy(idx_hbm.at[pl.ds(tile0, per_w)], idx_all)

        def gather(i, rows_v):
            pltpu.async_copy(
                tab_hbm.at[idx_all.at[jnp.minimum(i, lastl)]], rows_v, sem_r)

        for b in range(_NRING):
            gather(b, rows_bufs[b])

        iota = lax.iota(jnp.int32, 16)

        def wr_dst(g, chi):
            tile = tile0 + g * _NBUF
            return out_hbm.at[tile // n_bh, chi,
                              pl.ds(lax.rem(tile, n_bh), _NBUF)]

        def do_group(g, stage_v, ring0, drain):
            base = g * _NBUF
            @pl.when(drain)
            def _drain():
                for chi in range(dim_hi):
                    pltpu.make_async_copy(
                        stage_v.at[chi], wr_dst(g - 2, chi), sem_o).wait()

            for b in range(_NBUF):
                rows_v = rows_bufs[ring0 + b]
                pltpu.make_async_copy(
                    tab_hbm.at[idx_all.at[jnp.minimum(base + b, lastl)]],
                    rows_v, sem_r).wait()
                for chi in range(dim_hi):
                    for clo in range(8):
                        c = chi * 8 + clo
                        cvec = jnp.full((16,), c, jnp.int32)
                        vals = [
                            plsc.load_gather(rows_v, [iota + lg * 16, cvec])
                            for lg in range(_LANE // 16)
                        ]
                        for lg in range(_LANE // 16):
                            stage_v[chi, b, clo, pl.ds(lg * 16, 16)] = vals[lg]
                gather(base + _NRING + b, rows_v)
            for chi in range(dim_hi):
                pltpu.async_copy(stage_v.at[chi], wr_dst(g, chi), sem_o)

        def pair(gp, carry):
            do_group(2 * gp, stages[0], 0, gp > 0)
            do_group(2 * gp + 1, stages[1], _NBUF, gp > 0)
            return carry

        lax.fori_loop(0, n_groups // 2, pair, 0)

        for b in range(_NRING):
            pltpu.make_async_copy(
                tab_hbm.at[idx_all.at[lastl]], rows_bufs[b], sem_r).wait()
        for par in range(2):
            g = n_groups - 2 + par
            for chi in range(dim_hi):
                pltpu.make_async_copy(
                    stages[par].at[chi], wr_dst(g, chi), sem_o).wait()

    return k(idx_tiles, scaled_lin)


def kernel(x, table, freq_scale):
    batch, hist = x.shape
    dim = table.shape[1]
    tab_wide = _scale_table_tc(table.T, freq_scale)
    tab_rows = tab_wide.reshape(-1, dim)
    idx_tiles = _repack_idx_tc(x.T)
    idx_flat = idx_tiles.reshape(-1, _LANE)
    out5 = _sc_gather(idx_flat, tab_rows, hist, batch, dim)
    return out5.transpose(2, 4, 0, 1, 3).reshape(batch, hist, dim)
